# baseline (speedup 1.0000x reference)
import functools

import jax
import jax.numpy as jnp
from jax import lax
from jax.experimental import pallas as pl
from jax.experimental.pallas import tpu as pltpu
from jax.experimental.pallas import tpu_sc as plsc

DIM = 64
BATCH = 4096
MARGIN = 1.0
NC = 2
NS = 16
NW = NC * NS
BPW = BATCH // NW
LANES = 16
GROUPS = BPW // LANES
PACK = 2 * DIM

_GATHER_DNUMS = lax.GatherDimensionNumbers(
    offset_dims=(), collapsed_slice_dims=(0,), start_index_map=(0,))


def _shuffle16(v, perm):
    return lax.gather(v, perm, _GATHER_DNUMS, slice_sizes=(1,),
                      mode=lax.GatherScatterMode.PROMISE_IN_BOUNDS)


def _lane_total(v, perms):
    for perm in perms:
        v = v + _shuffle16(v, perm)
    return v


def _sqrt16(x):
    x = jnp.maximum(x, jnp.float32(1e-12))
    i = lax.bitcast_convert_type(x, jnp.int32)
    i = jnp.int32(0x5F3759DF) - lax.shift_right_logical(i, 1)
    y = lax.bitcast_convert_type(i, jnp.float32)
    for _ in range(3):
        y = y * (jnp.float32(1.5) - jnp.float32(0.5) * x * y * y)
    return x * y


_MESH = plsc.VectorSubcoreMesh(core_axis_name="c", subcore_axis_name="s")


@functools.partial(
    pl.kernel,
    out_type=jax.ShapeDtypeStruct((NW, LANES), jnp.float32),
    mesh=_MESH,
    scratch_types=[
        pltpu.VMEM((6, BPW), jnp.int32),
        pltpu.VMEM((BPW, PACK), jnp.float32),
        pltpu.VMEM((BPW, PACK), jnp.float32),
        pltpu.VMEM((BPW, PACK), jnp.float32),
        pltpu.VMEM((BPW, PACK), jnp.float32),
        pltpu.VMEM((BPW, PACK), jnp.float32),
        pltpu.VMEM((BPW, PACK), jnp.float32),
        pltpu.VMEM((LANES,), jnp.float32),
        pltpu.SemaphoreType.DMA,
    ],
)
def _transe_sc(ent_hbm, rel_hbm, idx_hbm, out_hbm,
               idx_v, hp_v, rp_v, tp_v, hn_v, rn_v, tn_v, loss_v,
               sem):
    wid = lax.axis_index("s") * NC + lax.axis_index("c")

    pltpu.sync_copy(idx_hbm.at[wid], idx_v)

    tables = (ent_hbm, rel_hbm, ent_hbm, ent_hbm, rel_hbm, ent_hbm)
    dsts = (hp_v, rp_v, tp_v, hn_v, rn_v, tn_v)

    def fetch_body(g, carry):
        base = g * LANES
        for j in range(6):
            vec = idx_v[j, pl.ds(base, LANES)]
            for b in range(LANES):
                i = vec[b]
                if j in (1, 4):
                    src = tables[j].at[i >> 1]
                else:
                    src = tables[j].at[i >> 4, (i >> 1) & 7]
                pltpu.async_copy(src, dsts[j].at[base + b], sem)
        return carry

    lax.fori_loop(0, GROUPS, fetch_body, 0)

    def drain_body(e, carry):
        for j in range(6):
            dummy = tables[j].at[0] if j in (1, 4) else tables[j].at[0, 0]
            pltpu.make_async_copy(dummy, dsts[j].at[e], sem).wait()
        return carry

    lax.fori_loop(0, BPW, drain_body, 0)

    iota = lax.iota(jnp.int32, LANES)
    zeros = jnp.zeros((LANES,), jnp.float32)
    perms = [(iota ^ k)[:, None] for k in (8, 4, 2, 1)]

    def group_body(g, loss_vec):
        base = g * LANES
        vecs = [idx_v[j, pl.ds(base, LANES)] for j in range(6)]
        d2p = zeros
        d2n = zeros
        for b in range(LANES):
            e = base + b
            offs = [(vecs[j][b] & 1) * DIM for j in range(6)]
            accp = zeros
            accn = zeros
            for c in range(DIM // LANES):
                dp = (hp_v[e, pl.ds(offs[0] + c * LANES, LANES)]
                      + rp_v[e, pl.ds(offs[1] + c * LANES, LANES)]
                      - tp_v[e, pl.ds(offs[2] + c * LANES, LANES)])
                accp = accp + dp * dp
                dn = (hn_v[e, pl.ds(offs[3] + c * LANES, LANES)]
                      + rn_v[e, pl.ds(offs[4] + c * LANES, LANES)]
                      - tn_v[e, pl.ds(offs[5] + c * LANES, LANES)])
                accn = accn + dn * dn
            lane = iota == b
            d2p = jnp.where(lane, _lane_total(accp, perms), d2p)
            d2n = jnp.where(lane, _lane_total(accn, perms), d2n)
        dpos = _sqrt16(d2p)
        dneg = _sqrt16(d2n)
        return loss_vec + jnp.maximum(jnp.float32(MARGIN) + dpos - dneg,
                                      jnp.float32(0.0))

    loss_v[...] = lax.fori_loop(0, GROUPS, group_body, zeros)
    pltpu.sync_copy(loss_v, out_hbm.at[wid])


def kernel(ent_emb, rel_emb, t_batch):
    idx = jnp.transpose(t_batch.astype(jnp.int32), (1, 2, 0))
    idx = jnp.transpose(idx.reshape(6, NW, BPW), (1, 0, 2))
    ent_pack = ent_emb.reshape(ent_emb.shape[0] // 16, 8, PACK)
    rel_pack = rel_emb.reshape(rel_emb.shape[0] // 2, PACK)
    partial = _transe_sc(ent_pack, rel_pack, idx)
    return jnp.sum(partial)


# speedup vs baseline: 2.4325x; 2.4325x over previous
import functools

import jax
import jax.numpy as jnp
from jax import lax
from jax.experimental import pallas as pl
from jax.experimental.pallas import tpu as pltpu
from jax.experimental.pallas import tpu_sc as plsc

DIM = 64
BATCH = 4096
MARGIN = 1.0
NC = 2
NS = 16
NW = NC * NS
BPW = BATCH // NW
LANES = 16
GROUPS = BPW // LANES
PACK = 2 * DIM

_GATHER_DNUMS = lax.GatherDimensionNumbers(
    offset_dims=(), collapsed_slice_dims=(0,), start_index_map=(0,))


def _shuffle16(v, perm):
    return lax.gather(v, perm, _GATHER_DNUMS, slice_sizes=(1,),
                      mode=lax.GatherScatterMode.PROMISE_IN_BOUNDS)


def _lane_total(v, perms):
    for perm in perms:
        v = v + _shuffle16(v, perm)
    return v


def _sqrt16(x):
    x = jnp.maximum(x, jnp.float32(1e-12))
    i = lax.bitcast_convert_type(x, jnp.int32)
    i = jnp.int32(0x5F3759DF) - lax.shift_right_logical(i, 1)
    y = lax.bitcast_convert_type(i, jnp.float32)
    for _ in range(3):
        y = y * (jnp.float32(1.5) - jnp.float32(0.5) * x * y * y)
    return x * y


_MESH = plsc.VectorSubcoreMesh(core_axis_name="c", subcore_axis_name="s")


@functools.partial(
    pl.kernel,
    out_type=jax.ShapeDtypeStruct((NW, LANES), jnp.float32),
    mesh=_MESH,
    scratch_types=[
        pltpu.VMEM((6, BPW), jnp.int32),
        pltpu.VMEM((BPW, DIM), jnp.float32),
        pltpu.VMEM((BPW, DIM), jnp.float32),
        pltpu.VMEM((BPW, DIM), jnp.float32),
        pltpu.VMEM((BPW, DIM), jnp.float32),
        pltpu.VMEM((BPW, DIM), jnp.float32),
        pltpu.VMEM((BPW, DIM), jnp.float32),
        pltpu.VMEM((LANES,), jnp.float32),
        pltpu.SemaphoreType.DMA,
    ],
)
def _transe_sc(ent_hbm, rel_hbm, idx_hbm, out_hbm,
               idx_v, hp_v, rp_v, tp_v, hn_v, rn_v, tn_v, loss_v,
               sem):
    wid = lax.axis_index("s") * NC + lax.axis_index("c")

    pltpu.sync_copy(idx_hbm.at[wid], idx_v)

    tables = (ent_hbm, rel_hbm, ent_hbm, ent_hbm, rel_hbm, ent_hbm)
    dsts = (hp_v, rp_v, tp_v, hn_v, rn_v, tn_v)

    def fetch_body(g, carry):
        base = g * LANES
        for j in range(6):
            vec = idx_v[j, pl.ds(base, LANES)]
            for b in range(LANES):
                i = vec[b]
                pltpu.async_copy(
                    tables[j].at[i >> 3, i & 7], dsts[j].at[base + b], sem)
        return carry

    lax.fori_loop(0, GROUPS, fetch_body, 0)

    def drain_body(e, carry):
        for j in range(6):
            pltpu.make_async_copy(
                tables[j].at[0, 0], dsts[j].at[e], sem).wait()
        return carry

    lax.fori_loop(0, BPW, drain_body, 0)

    iota = lax.iota(jnp.int32, LANES)
    zeros = jnp.zeros((LANES,), jnp.float32)
    perms = [(iota ^ k)[:, None] for k in (8, 4, 2, 1)]

    def group_body(g, loss_vec):
        def elem_body(b, carry):
            d2p, d2n = carry
            e = g * LANES + b
            accp = zeros
            accn = zeros
            for c in range(DIM // LANES):
                sl = pl.ds(c * LANES, LANES)
                dp = hp_v[e, sl] + rp_v[e, sl] - tp_v[e, sl]
                accp = accp + dp * dp
                dn = hn_v[e, sl] + rn_v[e, sl] - tn_v[e, sl]
                accn = accn + dn * dn
            lane = iota == b
            d2p = jnp.where(lane, _lane_total(accp, perms), d2p)
            d2n = jnp.where(lane, _lane_total(accn, perms), d2n)
            return d2p, d2n

        d2p, d2n = lax.fori_loop(0, LANES, elem_body, (zeros, zeros))
        dpos = _sqrt16(d2p)
        dneg = _sqrt16(d2n)
        return loss_vec + jnp.maximum(jnp.float32(MARGIN) + dpos - dneg,
                                      jnp.float32(0.0))

    loss_v[...] = lax.fori_loop(0, GROUPS, group_body, zeros)
    pltpu.sync_copy(loss_v, out_hbm.at[wid])


def kernel(ent_emb, rel_emb, t_batch):
    idx = jnp.transpose(t_batch.astype(jnp.int32), (1, 2, 0))
    idx = jnp.transpose(idx.reshape(6, NW, BPW), (1, 0, 2))
    ent3 = ent_emb.reshape(ent_emb.shape[0] // 8, 8, DIM)
    rel3 = rel_emb.reshape(rel_emb.shape[0] // 8, 8, DIM)
    partial = _transe_sc(ent3, rel3, idx)
    return jnp.sum(partial)
